# baseline (speedup 1.0000x reference)
import jax
import jax.numpy as jnp
from jax import lax
from jax.experimental import pallas as pl
from jax.experimental.pallas import tpu as pltpu
from jax.experimental.pallas import tpu_sc as plsc

B, L, D = 4096, 50, 128
NC, NS = 2, 16
NW = NC * NS
BPW = B // NW
NBUF = 5

SELU_ALPHA = 1.6732632423543772
SELU_SCALE = 1.0507009873554805


def _sc_cbow_body(idx_hbm, table_hbm, out_hbm, idx_v, acc_v,
                  b0, b1, b2, b3, b4, s0, s1, s2, s3, s4):
    bufs = (b0, b1, b2, b3, b4)
    sems = (s0, s1, s2, s3, s4)
    wid = lax.axis_index("s") * NC + lax.axis_index("c")
    base = wid * BPW
    pltpu.sync_copy(idx_hbm.at[wid], idx_v)

    def start(c, buf, sem):
        pltpu.async_copy(table_hbm.at[idx_v.at[c]], buf, sem)

    def wait(buf, sem):
        pltpu.make_async_copy(table_hbm.at[pl.ds(0, BPW), :], buf, sem).wait()

    def zero_rows(r, _):
        z = jnp.zeros((16,), jnp.float32)
        for d in range(D // 16):
            acc_v[r, pl.ds(d * 16, 16)] = z
        return 0

    lax.fori_loop(0, BPW, zero_rows, 0)

    for c in range(NBUF - 1):
        start(c, bufs[c], sems[c])

    def accum(buf):
        def body(r2, _):
            r = 2 * r2
            for rr in range(2):
                for d in range(D // 16):
                    sl = pl.ds(d * 16, 16)
                    plsc.addupdate(acc_v.at[r + rr, sl], buf[r + rr, sl])
            return 0
        lax.fori_loop(0, BPW // 2, body, 0)

    def step(i, _):
        for b in range(NBUF):
            c = i * NBUF + b
            nb = (b + NBUF - 1) % NBUF

            @pl.when(c + NBUF - 1 < L)
            def _(c=c, nb=nb):
                start(c + NBUF - 1, bufs[nb], sems[nb])

            wait(bufs[b], sems[b])
            accum(bufs[b])
        return 0

    lax.fori_loop(0, L // NBUF, step, 0)
    for b in range(L % NBUF):
        wait(bufs[b], sems[b])
        accum(bufs[b])
    pltpu.sync_copy(acc_v, out_hbm.at[pl.ds(base, BPW), :])


def _sc_cbow(idx_r, table):
    mesh = plsc.VectorSubcoreMesh(core_axis_name="c", subcore_axis_name="s")
    return pl.kernel(
        _sc_cbow_body,
        out_type=jax.ShapeDtypeStruct((B, D), jnp.float32),
        mesh=mesh,
        scratch_types=(
            [pltpu.VMEM((L, BPW), jnp.int32),
             pltpu.VMEM((BPW, D), jnp.float32)]
            + [pltpu.VMEM((BPW, D), jnp.float32)] * NBUF
            + [pltpu.SemaphoreType.DMA] * NBUF
        ),
    )(idx_r, table)


def _tc_proj_body(y_ref, w_ref, b_ref, o_ref):
    y = y_ref[...]
    s = jnp.where(y > 0, y, SELU_ALPHA * (jnp.exp(y) - 1.0)) * SELU_SCALE
    o_ref[...] = (
        lax.dot_general(s, w_ref[...], (((1,), (1,)), ((), ())),
                        preferred_element_type=jnp.float32)
        + b_ref[...]
    )


def _tc_proj(y, W, b2d):
    nblk = 1
    blk = B // nblk
    return pl.pallas_call(
        _tc_proj_body,
        grid=(nblk,),
        in_specs=[
            pl.BlockSpec((blk, D), lambda i: (i, 0)),
            pl.BlockSpec((D, D), lambda i: (0, 0)),
            pl.BlockSpec((1, D), lambda i: (0, 0)),
        ],
        out_specs=pl.BlockSpec((blk, D), lambda i: (i, 0)),
        out_shape=jax.ShapeDtypeStruct((B, D), jnp.float32),
    )(y, W, b2d)


def kernel(input_text, table, W, b):
    idx = input_text.reshape(B, L).astype(jnp.int32)
    idx_r = idx.reshape(NW, BPW, L).transpose(0, 2, 1)
    y = _sc_cbow(idx_r, table)
    return _tc_proj(y, W, b.reshape(1, D))


# speedup vs baseline: 1.0120x; 1.0120x over previous
import jax
import jax.numpy as jnp
from jax import lax
from jax.experimental import pallas as pl
from jax.experimental.pallas import tpu as pltpu
from jax.experimental.pallas import tpu_sc as plsc

B, L, D = 4096, 50, 128
NC, NS = 2, 16
NW = NC * NS
BPW = B // NW
NBUF = 5

SELU_ALPHA = 1.6732632423543772
SELU_SCALE = 1.0507009873554805


def _sc_cbow_body(idx_hbm, table_hbm, out_hbm, idx_v, acc_v,
                  b0, b1, b2, b3, b4, s0, s1, s2, s3, s4):
    bufs = (b0, b1, b2, b3, b4)
    sems = (s0, s1, s2, s3, s4)
    wid = lax.axis_index("s") * NC + lax.axis_index("c")
    base = wid * BPW
    pltpu.sync_copy(idx_hbm.at[wid], idx_v)

    def start(c, buf, sem):
        pltpu.async_copy(table_hbm.at[idx_v.at[c]], buf, sem)

    def wait(buf, sem):
        pltpu.make_async_copy(table_hbm.at[pl.ds(0, BPW), :], buf, sem).wait()

    for c in range(NBUF - 1):
        start(c, bufs[c], sems[c])

    def zero_rows(r, _):
        z = jnp.zeros((16,), jnp.float32)
        for d in range(D // 16):
            acc_v[r, pl.ds(d * 16, 16)] = z
        return 0

    lax.fori_loop(0, BPW, zero_rows, 0)

    def accum(buf):
        def body(r2, _):
            r = 2 * r2
            for rr in range(2):
                for d in range(D // 16):
                    sl = pl.ds(d * 16, 16)
                    plsc.addupdate(acc_v.at[r + rr, sl], buf[r + rr, sl])
            return 0
        lax.fori_loop(0, BPW // 2, body, 0)

    def step(i, _):
        for b in range(NBUF):
            c = i * NBUF + b
            nb = (b + NBUF - 1) % NBUF

            @pl.when(c + NBUF - 1 < L)
            def _(c=c, nb=nb):
                start(c + NBUF - 1, bufs[nb], sems[nb])

            wait(bufs[b], sems[b])
            accum(bufs[b])
        return 0

    lax.fori_loop(0, L // NBUF, step, 0)
    for b in range(L % NBUF):
        wait(bufs[b], sems[b])
        accum(bufs[b])
    pltpu.sync_copy(acc_v, out_hbm.at[pl.ds(base, BPW), :])


def _sc_cbow(idx_r, table):
    mesh = plsc.VectorSubcoreMesh(core_axis_name="c", subcore_axis_name="s")
    return pl.kernel(
        _sc_cbow_body,
        out_type=jax.ShapeDtypeStruct((B, D), jnp.float32),
        mesh=mesh,
        scratch_types=(
            [pltpu.VMEM((L, BPW), jnp.int32),
             pltpu.VMEM((BPW, D), jnp.float32)]
            + [pltpu.VMEM((BPW, D), jnp.float32)] * NBUF
            + [pltpu.SemaphoreType.DMA] * NBUF
        ),
    )(idx_r, table)


def _tc_proj_body(y_ref, w_ref, b_ref, o_ref):
    y = y_ref[...]
    s = jnp.where(y > 0, y, SELU_ALPHA * (jnp.exp(y) - 1.0)) * SELU_SCALE
    o_ref[...] = (
        lax.dot_general(s, w_ref[...], (((1,), (1,)), ((), ())),
                        preferred_element_type=jnp.float32)
        + b_ref[...]
    )


def _tc_proj(y, W, b2d):
    nblk = 2
    blk = B // nblk
    return pl.pallas_call(
        _tc_proj_body,
        grid=(nblk,),
        in_specs=[
            pl.BlockSpec((blk, D), lambda i: (i, 0)),
            pl.BlockSpec((D, D), lambda i: (0, 0)),
            pl.BlockSpec((1, D), lambda i: (0, 0)),
        ],
        out_specs=pl.BlockSpec((blk, D), lambda i: (i, 0)),
        out_shape=jax.ShapeDtypeStruct((B, D), jnp.float32),
    )(y, W, b2d)


def kernel(input_text, table, W, b):
    idx = input_text.reshape(B, L).astype(jnp.int32)
    idx_r = idx.reshape(NW, BPW, L).transpose(0, 2, 1)
    y = _sc_cbow(idx_r, table)
    return _tc_proj(y, W, b.reshape(1, D))
